# Initial kernel scaffold; baseline (speedup 1.0000x reference)
import jax
import jax.numpy as jnp
from jax.experimental import pallas as pl


def kernel(hidden_states, Wg, W1, W2, W3):
    raise NotImplementedError("write your pallas kernel here")


# speedup vs baseline: 1.1174x; 1.1174x over previous
import functools

import jax
import jax.numpy as jnp
from jax.experimental import pallas as pl
from jax.experimental.pallas import tpu as pltpu

T = 512
D = 2048
F = 7168
E = 8
EPAD = 128
BLK_F = 512


def _gate_body(logits_ref, ew_ref):
    lane = jax.lax.broadcasted_iota(jnp.int32, (T, EPAD), 1)
    valid = lane < E
    neg_inf = jnp.float32(-jnp.inf)
    logits = jnp.where(valid, logits_ref[...], neg_inf)
    lmax = jnp.max(logits, axis=1, keepdims=True)
    unnorm = jnp.exp(logits - lmax)
    p = unnorm / jnp.sum(unnorm, axis=1, keepdims=True)
    p = jnp.where(valid, p, neg_inf)
    m1 = jnp.max(p, axis=1, keepdims=True)
    i1 = jnp.min(jnp.where(p == m1, lane, EPAD), axis=1, keepdims=True)
    oh1 = lane == i1
    p2 = jnp.where(oh1, neg_inf, p)
    m2 = jnp.max(p2, axis=1, keepdims=True)
    i2 = jnp.min(jnp.where(p2 == m2, lane, EPAD), axis=1, keepdims=True)
    oh2 = lane == i2
    denom = m1 + m2
    ew = jnp.where(oh1, m1, 0.0) + jnp.where(oh2, m2, 0.0)
    ew_ref[...] = ew / denom


def _gate(logits_pad):
    return pl.pallas_call(
        _gate_body,
        out_shape=jax.ShapeDtypeStruct((T, EPAD), jnp.float32),
    )(logits_pad)


def _moe_body(x_ref, w1_ref, w3_ref, w2_ref, ew_ref, o_ref, xb_ref):
    e = pl.program_id(0)
    f = pl.program_id(1)

    @pl.when((e == 0) & (f == 0))
    def _init():
        o_ref[...] = jnp.zeros_like(o_ref)
        xb_ref[...] = x_ref[...].astype(jnp.bfloat16)

    xb = xb_ref[...]
    w1 = w1_ref[0].astype(jnp.bfloat16)
    w3 = w3_ref[0].astype(jnp.bfloat16)
    w2 = w2_ref[0].astype(jnp.bfloat16)
    h1 = jax.lax.dot_general(
        xb, w1, (((1,), (0,)), ((), ())), preferred_element_type=jnp.float32)
    h3 = jax.lax.dot_general(
        xb, w3, (((1,), (0,)), ((), ())), preferred_element_type=jnp.float32)
    g = (h1 * jax.lax.logistic(h1)) * h3
    g = g * ew_ref[0]
    o_ref[...] += jax.lax.dot_general(
        g.astype(jnp.bfloat16), w2, (((1,), (0,)), ((), ())),
        preferred_element_type=jnp.float32)


def _moe(x, w1, w3, w2, ew3):
    nf = F // BLK_F
    return pl.pallas_call(
        _moe_body,
        grid=(E, nf),
        in_specs=[
            pl.BlockSpec((T, D), lambda e, f: (0, 0)),
            pl.BlockSpec((1, D, BLK_F), lambda e, f: (e, 0, f)),
            pl.BlockSpec((1, D, BLK_F), lambda e, f: (e, 0, f)),
            pl.BlockSpec((1, BLK_F, D), lambda e, f: (e, f, 0)),
            pl.BlockSpec((1, T, 1), lambda e, f: (e, 0, 0)),
        ],
        out_specs=pl.BlockSpec((T, D), lambda e, f: (0, 0)),
        out_shape=jax.ShapeDtypeStruct((T, D), jnp.float32),
        scratch_shapes=[pltpu.VMEM((T, D), jnp.bfloat16)],
        compiler_params=pltpu.CompilerParams(
            dimension_semantics=("arbitrary", "arbitrary")),
    )(x, w1, w3, w2, ew3)


@jax.jit
def kernel(hidden_states, Wg, W1, W2, W3):
    router_logits = hidden_states @ Wg
    logits_pad = jnp.pad(router_logits, ((0, 0), (0, EPAD - E)),
                         constant_values=-jnp.inf)
    ew_t = _gate(logits_pad)
    ew3 = jnp.transpose(ew_t[:, :E])[:, :, None]
    return _moe(hidden_states, W1, W3, W2, ew3)
